# baseline (speedup 1.0000x reference)
import functools

import jax
import jax.numpy as jnp
from jax import lax
from jax.experimental import pallas as pl
from jax.experimental.pallas import tpu as pltpu
from jax.experimental.pallas import tpu_sc as plsc

N_V = 10000
N_HE = 10000
D_IN = 128
D = 128
NP = 10240
NC = 2
NS = 16
NW = NC * NS
BS = 128
EB = 80
EP = NW * EB * BS


def _mm_body(x_ref, w_ref, b_ref, o_ref):
    o_ref[...] = (
        jnp.dot(x_ref[...], w_ref[...], preferred_element_type=jnp.float32)
        + b_ref[...]
    )


def _deg_body(dh_ref, o_ref):
    o_ref[...] = jnp.sum(dh_ref[...], axis=0)


def _comb1_body(p_ref, d_ref, o_ref):
    deg = jnp.maximum(d_ref[...], 1.0)
    o_ref[...] = (p_ref[0] + p_ref[1]) / deg


def _comb2_body(p_ref, d_ref, o_ref):
    deg = jnp.maximum(d_ref[0:N_V], 1.0)
    y = (p_ref[0, 0:N_V] + p_ref[1, 0:N_V]) / deg
    o_ref[...] = jnp.where(y >= 0, y, 0.01 * y)


def _sc_pass_body(feat_hbm, gidx_hbm, sidx_hbm, outf_hbm, outd_hbm,
                  gw0, gw1, sw0, sw1, buf0, buf1, hist, acc_sh,
                  isem0, isem1, gsem0, gsem1, ssem0, ssem1):
    c = lax.axis_index("c")
    s = lax.axis_index("s")
    rows_per_tile = NP // NS

    z16 = jnp.zeros((16,), jnp.float32)
    ones16 = jnp.ones((16,), jnp.float32)

    def zrow(i, carry):
        for k in range(D // 16):
            buf0[i, pl.ds(k * 16, 16)] = z16
        return carry

    lax.fori_loop(0, BS, zrow, 0)

    def zhist(i, carry):
        hist[pl.ds(i * 16, 16)] = z16
        return carry

    lax.fori_loop(0, NP // 16, zhist, 0)

    for k in range(rows_per_tile // BS):
        pltpu.sync_copy(buf0, acc_sh.at[pl.ds(s * rows_per_tile + k * BS, BS)])
    plsc.subcore_barrier()

    gws = (gw0, gw1)
    sws = (sw0, sw1)
    bufs = (buf0, buf1)
    isems = (isem0, isem1)
    gsems = (gsem0, gsem1)
    ssems = (ssem0, ssem1)

    pltpu.sync_copy(gidx_hbm.at[c, s, 0], gw0)
    pltpu.sync_copy(sidx_hbm.at[c, s, 0], sw0)
    pltpu.async_copy(feat_hbm.at[gw0], buf0, gsem0)
    pltpu.async_copy(gidx_hbm.at[c, s, 1], gw1, isem1)
    pltpu.async_copy(sidx_hbm.at[c, s, 1], sw1, isem1)

    def half(j, p):
        q = 1 - p
        pltpu.make_async_copy(gidx_hbm.at[c, s, 0], gws[q], isems[q]).wait()
        pltpu.make_async_copy(sidx_hbm.at[c, s, 0], sws[q], isems[q]).wait()
        pltpu.async_copy(feat_hbm.at[gws[q]], bufs[q], gsems[q])
        pltpu.make_async_copy(feat_hbm.at[gws[p]], bufs[p], gsems[p]).wait()
        pltpu.async_copy(bufs[p], acc_sh.at[sws[p]], ssems[p], add=True)
        for k in range(BS // 16):
            si = sws[p][pl.ds(k * 16, 16)]
            plsc.addupdate_scatter(hist, [si], ones16)
        pltpu.make_async_copy(bufs[p], acc_sh.at[sws[p]], ssems[p]).wait()
        pltpu.async_copy(gidx_hbm.at[c, s, j + 2], gws[p], isems[p])
        pltpu.async_copy(sidx_hbm.at[c, s, j + 2], sws[p], isems[p])

    def body(i, carry):
        half(2 * i, 0)
        half(2 * i + 1, 1)
        return carry

    lax.fori_loop(0, EB // 2, body, 0)

    pltpu.make_async_copy(feat_hbm.at[gw0], buf0, gsem0).wait()
    pltpu.make_async_copy(gidx_hbm.at[c, s, 0], gw1, isem1).wait()
    pltpu.make_async_copy(sidx_hbm.at[c, s, 0], sw1, isem1).wait()
    plsc.subcore_barrier()

    pltpu.sync_copy(acc_sh.at[pl.ds(s * rows_per_tile, rows_per_tile)],
                    outf_hbm.at[c, pl.ds(s * rows_per_tile, rows_per_tile)])
    pltpu.sync_copy(hist, outd_hbm.at[c, s])


_sc_pass = functools.partial(
    pl.kernel,
    mesh=plsc.VectorSubcoreMesh(core_axis_name="c", subcore_axis_name="s"),
    compiler_params=pltpu.CompilerParams(needs_layout_passes=False),
    out_type=[
        jax.ShapeDtypeStruct((NC, NP, D), jnp.float32),
        jax.ShapeDtypeStruct((NC, NS, NP), jnp.float32),
    ],
    scratch_types=[
        pltpu.VMEM((BS,), jnp.int32),
        pltpu.VMEM((BS,), jnp.int32),
        pltpu.VMEM((BS,), jnp.int32),
        pltpu.VMEM((BS,), jnp.int32),
        pltpu.VMEM((BS, D), jnp.float32),
        pltpu.VMEM((BS, D), jnp.float32),
        pltpu.VMEM((NP,), jnp.float32),
        pltpu.VMEM_SHARED((NP, D), jnp.float32),
        pltpu.SemaphoreType.DMA,
        pltpu.SemaphoreType.DMA,
        pltpu.SemaphoreType.DMA,
        pltpu.SemaphoreType.DMA,
        pltpu.SemaphoreType.DMA,
        pltpu.SemaphoreType.DMA,
    ],
)(_sc_pass_body)


def _sum_hists(dh):
    return pl.pallas_call(
        _deg_body,
        out_shape=jax.ShapeDtypeStruct((NP,), jnp.float32),
    )(dh.reshape(NW, NP)).reshape(NP, 1)


def kernel(X, edge_index, W, b):
    X = X.astype(jnp.float32)
    W = W.astype(jnp.float32)
    b = b.astype(jnp.float32)

    X_pad = jnp.zeros((NP, D_IN), jnp.float32).at[:N_V].set(X)
    Xt = pl.pallas_call(
        _mm_body,
        out_shape=jax.ShapeDtypeStruct((NP, D), jnp.float32),
    )(X_pad, W, b[None, :])

    vid = edge_index[0].astype(jnp.int32)
    eid = edge_index[1].astype(jnp.int32)
    e_inc = vid.shape[0]
    pad = jnp.full((EP - e_inc,), NP - 1, jnp.int32)
    look = jnp.full((NC, NS, 2, BS), NP - 1, jnp.int32)
    vid_b = jnp.concatenate(
        [jnp.concatenate([vid, pad]).reshape(NC, NS, EB, BS), look], axis=2)
    eid_b = jnp.concatenate(
        [jnp.concatenate([eid, pad]).reshape(NC, NS, EB, BS), look], axis=2)

    p1, dh1 = _sc_pass(Xt, vid_b, eid_b)
    e_feat = pl.pallas_call(
        _comb1_body,
        out_shape=jax.ShapeDtypeStruct((NP, D), jnp.float32),
    )(p1, _sum_hists(dh1))

    p2, dh2 = _sc_pass(e_feat, eid_b, vid_b)
    out = pl.pallas_call(
        _comb2_body,
        out_shape=jax.ShapeDtypeStruct((N_V, D), jnp.float32),
    )(p2, _sum_hists(dh2))
    return out
